# baseline (speedup 1.0000x reference)
import functools

import jax
import jax.numpy as jnp
from jax import lax
from jax.experimental import pallas as pl
from jax.experimental.pallas import tpu as pltpu
from jax.experimental.pallas import tpu_sc as plsc

D = 64
_NUM_WORKERS = 32
_FPW = D // _NUM_WORKERS


def _sc_gather6(elf, erf, rtf, n_ent, n_rel,
                idx_s, idx_o, idx_ss, idx_so, idx_obs, idx_samp):
    B = idx_s.shape[0]
    mesh = plsc.VectorSubcoreMesh(core_axis_name="c", subcore_axis_name="s")
    out_t = tuple(jax.ShapeDtypeStruct((D, B), jnp.float32) for _ in range(6))

    @functools.partial(
        pl.kernel,
        mesh=mesh,
        out_type=out_t,
        scratch_types=[
            pltpu.VMEM((B,), jnp.int32),
            pltpu.VMEM((B,), jnp.int32),
            pltpu.VMEM((_FPW, B), jnp.float32),
            pltpu.SemaphoreType.DMA,
        ],
        compiler_params=pltpu.CompilerParams(use_tc_tiling_on_sc=False),
    )
    def k(el, er, rt, i_s, i_o, i_ss, i_so, i_ob, i_sp,
          o_s, o_o, o_ss, o_so, o_ob, o_sp, idx_v, idx2_v, rows_v, sem):
        wid = lax.axis_index("s") * 2 + lax.axis_index("c")
        ops = ((el, n_ent, i_s, o_s), (er, n_ent, i_o, o_o),
               (el, n_ent, i_ss, o_ss), (er, n_ent, i_so, o_so),
               (rt, n_rel, i_ob, o_ob), (rt, n_rel, i_sp, o_sp))
        for tab, n_rows, ih, oh in ops:
            pltpu.sync_copy(ih, idx_v)
            for f in range(_FPW):
                j = wid * _FPW + f

                def shift(t, _, j=j, n_rows=n_rows):
                    idx2_v[pl.ds(t * 16, 16)] = (
                        idx_v[pl.ds(t * 16, 16)] + j * n_rows)
                    return _

                lax.fori_loop(0, B // 16, shift, 0, unroll=8)
                pltpu.async_copy(tab.at[idx2_v], rows_v.at[f], sem).wait()
                pltpu.sync_copy(rows_v.at[f], oh.at[j])

    return k(elf, erf, rtf, idx_s, idx_o, idx_ss, idx_so, idx_obs, idx_samp)


def _logsig(x):
    return jnp.minimum(x, 0.0) - jnp.log1p(jnp.exp(-jnp.abs(x)))


def _tc_body(s, o, ss, so, ob, sp, w1t, w2t, bias,
             pred_ref, pobs_ref, psamp_ref, loss_ref):
    a = jnp.dot(w1t[...], s[...], preferred_element_type=jnp.float32)
    bo = jnp.dot(w2t[...], o[...], preferred_element_type=jnp.float32)
    c = jnp.dot(w1t[...], ss[...], preferred_element_type=jnp.float32)
    e = jnp.dot(w2t[...], so[...], preferred_element_type=jnp.float32)
    bb = bias[...]
    pred = jnp.tanh(a + bo + bb)
    pss = jnp.tanh(c + bo + bb)
    pso = jnp.tanh(a + e + bb)
    obv = ob[...]
    spv = sp[...]
    pos = jnp.sum(pred * obv, axis=0)
    neg = jnp.sum(pred * spv, axis=0)
    nss = jnp.sum(pss * obv, axis=0)
    nso = jnp.sum(pso * obv, axis=0)
    pred_ref[...] = pred
    pobs_ref[...] = jax.nn.sigmoid(pos)
    psamp_ref[...] = jax.nn.sigmoid(neg)
    part = -(jnp.sum(_logsig(pos)) + 2.0 * jnp.sum(_logsig(-neg))
             + 0.5 * jnp.sum(_logsig(-nss)) + 0.5 * jnp.sum(_logsig(-nso)))

    @pl.when(pl.program_id(0) == 0)
    def _():
        loss_ref[...] = jnp.zeros_like(loss_ref)

    loss_ref[...] += jnp.reshape(part, (1, 1))


def _tc_compute(s_t, o_t, ss_t, so_t, obs_t, samp_t, W1t, W2t, bcol,
                interpret=False):
    B = s_t.shape[1]
    BB = 2048
    nb = B // BB
    col = pl.BlockSpec((D, BB), lambda i: (0, i))
    full = pl.BlockSpec((D, D), lambda i: (0, 0))
    vec = pl.BlockSpec((BB,), lambda i: (i,))
    return pl.pallas_call(
        _tc_body,
        grid=(nb,),
        in_specs=[col, col, col, col, col, col, full, full,
                  pl.BlockSpec((D, 1), lambda i: (0, 0))],
        out_specs=[col, vec, vec, pl.BlockSpec((1, 1), lambda i: (0, 0))],
        out_shape=[
            jax.ShapeDtypeStruct((D, B), jnp.float32),
            jax.ShapeDtypeStruct((B,), jnp.float32),
            jax.ShapeDtypeStruct((B,), jnp.float32),
            jax.ShapeDtypeStruct((1, 1), jnp.float32),
        ],
        interpret=interpret,
    )(s_t, o_t, ss_t, so_t, obs_t, samp_t, W1t, W2t, bcol)


def kernel(subjects, objects, observed_relations, sampled_relations,
           sampled_subjects, sampled_objects,
           ent_left, ent_right, rel_table, W, b):
    idx_s = subjects.astype(jnp.int32)
    idx_o = objects.astype(jnp.int32)
    idx_ss = sampled_subjects.astype(jnp.int32)
    idx_so = sampled_objects.astype(jnp.int32)
    idx_obs = observed_relations[:, 0].astype(jnp.int32)
    idx_samp = sampled_relations[:, 0].astype(jnp.int32)

    n_ent = ent_left.shape[0]
    n_rel = rel_table.shape[0]
    elf = ent_left.T.reshape(-1)
    erf = ent_right.T.reshape(-1)
    rtf = rel_table.T.reshape(-1)

    s_t, o_t, ss_t, so_t, obs_t, samp_t = _sc_gather6(
        elf, erf, rtf, n_ent, n_rel,
        idx_s, idx_o, idx_ss, idx_so, idx_obs, idx_samp)

    Wt = W.T
    W1t = Wt[:, :D]
    W2t = Wt[:, D:]
    bcol = b.reshape(D, 1)
    pred_t, pobs, psamp, loss = _tc_compute(
        s_t, o_t, ss_t, so_t, obs_t, samp_t, W1t, W2t, bcol)
    return pred_t.T, loss[0, 0], pobs, psamp


# speedup vs baseline: 18.2251x; 18.2251x over previous
import functools

import jax
import jax.numpy as jnp
from jax import lax
from jax.experimental import pallas as pl
from jax.experimental.pallas import tpu as pltpu
from jax.experimental.pallas import tpu_sc as plsc

D = 64
_NUM_WORKERS = 32


def _sc_gather6(ent_left3, ent_right3, rel_table3,
                idx_s, idx_o, idx_ss, idx_so, idx_obs, idx_samp):
    B = idx_s.shape[0]
    bpw = B // _NUM_WORKERS
    mesh = plsc.VectorSubcoreMesh(core_axis_name="c", subcore_axis_name="s")
    out_t = tuple(jax.ShapeDtypeStruct((B, D), jnp.float32) for _ in range(6))

    @functools.partial(
        pl.kernel,
        mesh=mesh,
        out_type=out_t,
        scratch_types=[
            pltpu.VMEM((bpw,), jnp.int32),
            pltpu.VMEM((bpw, D), jnp.float32),
            pltpu.SemaphoreType.DMA,
        ],
        compiler_params=pltpu.CompilerParams(use_tc_tiling_on_sc=True),
    )
    def k(el, er, rt, i_s, i_o, i_ss, i_so, i_ob, i_sp,
          o_s, o_o, o_ss, o_so, o_ob, o_sp, idx_v, sel_v, sem):
        wid = lax.axis_index("s") * 2 + lax.axis_index("c")
        base = wid * bpw
        ops = ((el, i_s, o_s), (er, i_o, o_o), (el, i_ss, o_ss),
               (er, i_so, o_so), (rt, i_ob, o_ob), (rt, i_sp, o_sp))
        for tab, ih, oh in ops:
            pltpu.sync_copy(ih.at[pl.ds(base, bpw)], idx_v)

            def issue(t, _, tab=tab):
                v = idx_v[pl.ds(t * 16, 16)]
                blk = lax.shift_right_logical(v, 3)
                row = lax.bitwise_and(v, 7)
                for l in range(16):
                    pltpu.async_copy(tab.at[blk[l], row[l]],
                                     sel_v.at[t * 16 + l], sem)
                return _

            lax.fori_loop(0, bpw // 16, issue, 0)

            def drain(i, _, tab=tab):
                pltpu.make_async_copy(tab.at[0, 0], sel_v.at[0], sem).wait()
                return _

            lax.fori_loop(0, bpw, drain, 0)
            pltpu.sync_copy(sel_v, oh.at[pl.ds(base, bpw)])

    return k(ent_left3, ent_right3, rel_table3,
             idx_s, idx_o, idx_ss, idx_so, idx_obs, idx_samp)


def _logsig(x):
    return jnp.minimum(x, 0.0) - jnp.log1p(jnp.exp(-jnp.abs(x)))


def _tc_body(s, o, ss, so, ob, sp, w1, w2, bias,
             pred_ref, pobs_ref, psamp_ref, loss_ref):
    a = jnp.dot(s[...], w1[...], preferred_element_type=jnp.float32)
    bo = jnp.dot(o[...], w2[...], preferred_element_type=jnp.float32)
    c = jnp.dot(ss[...], w1[...], preferred_element_type=jnp.float32)
    e = jnp.dot(so[...], w2[...], preferred_element_type=jnp.float32)
    bb = bias[...]
    pred = jnp.tanh(a + bo + bb)
    pss = jnp.tanh(c + bo + bb)
    pso = jnp.tanh(a + e + bb)
    obv = ob[...]
    spv = sp[...]
    pos = jnp.sum(pred * obv, axis=-1)
    neg = jnp.sum(pred * spv, axis=-1)
    nss = jnp.sum(pss * obv, axis=-1)
    nso = jnp.sum(pso * obv, axis=-1)
    pred_ref[...] = pred
    pobs_ref[...] = jax.nn.sigmoid(pos)
    psamp_ref[...] = jax.nn.sigmoid(neg)
    part = -(jnp.sum(_logsig(pos)) + 2.0 * jnp.sum(_logsig(-neg))
             + 0.5 * jnp.sum(_logsig(-nss)) + 0.5 * jnp.sum(_logsig(-nso)))

    @pl.when(pl.program_id(0) == 0)
    def _():
        loss_ref[...] = jnp.zeros_like(loss_ref)

    loss_ref[...] += jnp.reshape(part, (1, 1))


def _tc_compute(s_emb, o_emb, ss_emb, so_emb, obs_emb, samp_emb, W1, W2, b2,
                interpret=False):
    B = s_emb.shape[0]
    BB = 2048
    nb = B // BB
    row = pl.BlockSpec((BB, D), lambda i: (i, 0))
    full = pl.BlockSpec((D, D), lambda i: (0, 0))
    vec = pl.BlockSpec((BB,), lambda i: (i,))
    return pl.pallas_call(
        _tc_body,
        grid=(nb,),
        in_specs=[row, row, row, row, row, row, full, full,
                  pl.BlockSpec((1, D), lambda i: (0, 0))],
        out_specs=[row, vec, vec, pl.BlockSpec((1, 1), lambda i: (0, 0))],
        out_shape=[
            jax.ShapeDtypeStruct((B, D), jnp.float32),
            jax.ShapeDtypeStruct((B,), jnp.float32),
            jax.ShapeDtypeStruct((B,), jnp.float32),
            jax.ShapeDtypeStruct((1, 1), jnp.float32),
        ],
        interpret=interpret,
    )(s_emb, o_emb, ss_emb, so_emb, obs_emb, samp_emb, W1, W2, b2)


def kernel(subjects, objects, observed_relations, sampled_relations,
           sampled_subjects, sampled_objects,
           ent_left, ent_right, rel_table, W, b):
    idx_s = subjects.astype(jnp.int32)
    idx_o = objects.astype(jnp.int32)
    idx_ss = sampled_subjects.astype(jnp.int32)
    idx_so = sampled_objects.astype(jnp.int32)
    idx_obs = observed_relations[:, 0].astype(jnp.int32)
    idx_samp = sampled_relations[:, 0].astype(jnp.int32)

    el3 = ent_left.reshape(ent_left.shape[0] // 8, 8, D)
    er3 = ent_right.reshape(ent_right.shape[0] // 8, 8, D)
    rt3 = rel_table.reshape(rel_table.shape[0] // 8, 8, D)

    s_emb, o_emb, ss_emb, so_emb, obs_emb, samp_emb = _sc_gather6(
        el3, er3, rt3, idx_s, idx_o, idx_ss, idx_so, idx_obs, idx_samp)

    W1 = W[:D]
    W2 = W[D:]
    b2 = b.reshape(1, D)
    pred, pobs, psamp, loss = _tc_compute(
        s_emb, o_emb, ss_emb, so_emb, obs_emb, samp_emb, W1, W2, b2)
    return pred, loss[0, 0], pobs, psamp
